# baseline (speedup 1.0000x reference)
import functools

import jax
import jax.numpy as jnp
from jax import lax
from jax.experimental import pallas as pl
from jax.experimental.pallas import tpu as pltpu
from jax.experimental.pallas import tpu_sc as plsc

_N = 10000
_E = 160000
_F = 256
_H = 256
_C = 64
_FH = 128
_NC = 2
_NS = 16
_K = 64
_NG = 160
_GP = 32
_NPH = _NG // _GP
_EP = _NS * _NG * _K
_NP = _N + 16
_ZR = 8
_WT = 10
_RW = _N // _WT

_BM = 1000
_MB = _N // _BM


def _gemm_in_body(x_ref, w_ref, o_ref):
    o_ref[...] = jnp.dot(x_ref[...], w_ref[...],
                         preferred_element_type=jnp.float32)


def _gemm_in(x, W):
    return pl.pallas_call(
        _gemm_in_body,
        grid=(_NC, _MB),
        in_specs=[
            pl.BlockSpec((_BM, _F), lambda c, i: (i, 0)),
            pl.BlockSpec((_F, _FH), lambda c, i: (0, c)),
        ],
        out_specs=pl.BlockSpec((_BM, _FH), lambda c, i: (c * _MB + i, 0)),
        out_shape=jax.ShapeDtypeStruct((_NC * _N, _FH), jnp.float32),
    )(x, W)


def _gemm_mid_body(z0_ref, z1_ref, b_ref, w_ref, o_ref):
    h0 = jnp.maximum(z0_ref[...] + b_ref[0:1, :], 0.0)
    h1 = jnp.maximum(z1_ref[...] + b_ref[1:2, :], 0.0)
    o_ref[...] = (
        jnp.dot(h0, w_ref[0:_FH, :], preferred_element_type=jnp.float32)
        + jnp.dot(h1, w_ref[_FH:, :], preferred_element_type=jnp.float32))


def _gemm_mid(z, b, W):
    b2 = b.reshape(_NC, _FH)
    return pl.pallas_call(
        _gemm_mid_body,
        grid=(_NC, _MB),
        in_specs=[
            pl.BlockSpec((_BM, _FH), lambda c, i: (i, 0)),
            pl.BlockSpec((_BM, _FH), lambda c, i: (_MB + i, 0)),
            pl.BlockSpec((_NC, _FH), lambda c, i: (0, 0)),
            pl.BlockSpec((_H, _FH), lambda c, i: (0, c)),
        ],
        out_specs=pl.BlockSpec((_BM, _FH), lambda c, i: (c * _MB + i, 0)),
        out_shape=jax.ShapeDtypeStruct((_NC * _N, _FH), jnp.float32),
    )(z, z, b2, W)


def _gemm_out_body(z0_ref, z1_ref, b_ref, w_ref, b3_ref, o_ref):
    h0 = jnp.maximum(z0_ref[...] + b_ref[0:1, :], 0.0)
    h1 = jnp.maximum(z1_ref[...] + b_ref[1:2, :], 0.0)
    o_ref[...] = (
        jnp.dot(h0, w_ref[0:_FH, :], preferred_element_type=jnp.float32)
        + jnp.dot(h1, w_ref[_FH:, :], preferred_element_type=jnp.float32)
        + b3_ref[...])


def _gemm_out(z, b, W3, b3):
    b2 = b.reshape(_NC, _FH)
    b3r = b3.reshape(1, _C)
    return pl.pallas_call(
        _gemm_out_body,
        grid=(_MB,),
        in_specs=[
            pl.BlockSpec((_BM, _FH), lambda i: (i, 0)),
            pl.BlockSpec((_BM, _FH), lambda i: (_MB + i, 0)),
            pl.BlockSpec((_NC, _FH), lambda i: (0, 0)),
            pl.BlockSpec((_H, _C), lambda i: (0, 0)),
            pl.BlockSpec((1, _C), lambda i: (0, 0)),
        ],
        out_specs=pl.BlockSpec((_BM, _C), lambda i: (i, 0)),
        out_shape=jax.ShapeDtypeStruct((_N, _C), jnp.float32),
    )(z, z, b2, W3, b3r)


_mesh = plsc.VectorSubcoreMesh(core_axis_name="c", subcore_axis_name="s")


@functools.partial(
    pl.kernel,
    mesh=_mesh,
    out_type=jax.ShapeDtypeStruct((_NC * _N, _FH), jnp.float32),
    scratch_types=[
        pltpu.VMEM((_GP, _K), jnp.int32),
        pltpu.VMEM((_GP, _K), jnp.int32),
        pltpu.VMEM((_K, _FH), jnp.float32),
        pltpu.VMEM((_K, _FH), jnp.float32),
        pltpu.VMEM((_K, _FH), jnp.float32),
        pltpu.VMEM((_K, _FH), jnp.float32),
        pltpu.VMEM((_ZR, _FH), jnp.float32),
        pltpu.VMEM_SHARED((_NP, _FH), jnp.float32),
        pltpu.SemaphoreType.DMA,
        pltpu.SemaphoreType.DMA,
        pltpu.SemaphoreType.DMA,
        pltpu.SemaphoreType.DMA,
        pltpu.SemaphoreType.DMA,
    ],
)
def _spmm(srcA_hbm, srcB_hbm, dst_hbm, y_hbm, out_hbm,
          src_v, dst_v, rows0_v, rows1_v, rows2_v, rows3_v, zero_v, accum,
          sem0, sem1, sem2, sem3, zsem):
    c = lax.axis_index("c")
    s = lax.axis_index("s")

    def _stage(p):
        @pl.when(c == 0)
        def _():
            pltpu.sync_copy(srcA_hbm.at[pl.ds(s * _NG + p * _GP, _GP)],
                            src_v)

        @pl.when(c == 1)
        def _():
            pltpu.sync_copy(srcB_hbm.at[pl.ds(s * _NG + p * _GP, _GP)],
                            src_v)

        pltpu.sync_copy(dst_hbm.at[pl.ds(s * _NG + p * _GP, _GP)], dst_v)

    bufs = (rows0_v, rows1_v, rows2_v, rows3_v)
    sems = (sem0, sem1, sem2, sem3)
    _NB = len(bufs)

    def _prime():
        for b in range(_NB):
            pltpu.async_copy(y_hbm.at[src_v.at[b]], bufs[b], sems[b])

    def _edge_phase():
        def _body(j, carry):
            g = _NB * j
            for b in range(_NB):
                pltpu.make_async_copy(y_hbm.at[src_v.at[g + b]], bufs[b],
                                      sems[b]).wait()
                pltpu.sync_copy(bufs[b], accum.at[dst_v.at[g + b]],
                                add=True)
                pltpu.async_copy(y_hbm.at[src_v.at[g + b + _NB]], bufs[b],
                                 sems[b])
            return carry
        n_full = (_GP - 2 * _NB) // _NB + 1
        lax.fori_loop(0, n_full, _body, 0)

        for g in range(_NB * n_full, _GP):
            b = g % _NB
            pltpu.make_async_copy(y_hbm.at[src_v.at[g]], bufs[b],
                                  sems[b]).wait()
            pltpu.sync_copy(bufs[b], accum.at[dst_v.at[g]], add=True)
            if g + _NB < _GP:
                pltpu.async_copy(y_hbm.at[src_v.at[g + _NB]], bufs[b],
                                 sems[b])

    _stage(0)

    _prime()

    @pl.when(s < _WT)
    def _():
        def _zb(i, carry):
            zero_v[i // 8, pl.ds((i % 8) * 16, 16)] = jnp.zeros(
                (16,), jnp.float32)
            return carry
        lax.fori_loop(0, _ZR * 8, _zb, 0)

        def _zc(i, carry):
            pltpu.async_copy(zero_v, accum.at[pl.ds(s * _RW + i * _ZR, _ZR)],
                             zsem)
            return carry
        lax.fori_loop(0, _RW // _ZR, _zc, 0)

        def _zw(i, carry):
            pltpu.make_async_copy(
                zero_v, accum.at[pl.ds(s * _RW, _ZR)], zsem).wait()
            return carry
        lax.fori_loop(0, _RW // _ZR, _zw, 0)

    plsc.subcore_barrier()

    _edge_phase()
    for p in range(1, _NPH):
        _stage(p)
        _prime()
        _edge_phase()

    plsc.subcore_barrier()

    @pl.when(s < _WT)
    def _():
        pltpu.sync_copy(accum.at[pl.ds(s * _RW, _RW)],
                        out_hbm.at[pl.ds(c * _N + s * _RW, _RW)])


def kernel(x, edge_index, W1, b1, W2, b2, W3, b3):
    pad = _EP - _E
    pad_iota = jnp.arange(pad, dtype=jnp.int32)
    pad2 = jnp.stack([pad_iota % _N, _N + (pad_iota % 16)])
    ei = jnp.concatenate([edge_index, pad2], axis=1)
    src = ei[0].reshape(_NS * _NG, _K)
    srcB = src + _N
    dst = ei[1].reshape(_NS * _NG, _K)

    y1 = _gemm_in(x, W1)
    z1 = _spmm(src, srcB, dst, y1)
    y2 = _gemm_mid(z1, b1, W2)
    z2 = _spmm(src, srcB, dst, y2)

    return _gemm_out(z2, b2, W3, b3)


# speedup vs baseline: 1.0078x; 1.0078x over previous
import functools

import jax
import jax.numpy as jnp
from jax import lax
from jax.experimental import pallas as pl
from jax.experimental.pallas import tpu as pltpu
from jax.experimental.pallas import tpu_sc as plsc

_N = 10000
_E = 160000
_F = 256
_H = 256
_C = 64
_FH = 128
_NC = 2
_NS = 16
_K = 64
_NG = 160
_GP = 32
_NPH = _NG // _GP
_EP = _NS * _NG * _K
_RWA = 632
_NP = _NS * _RWA
_RWL = 520
_ZR = 8

_BM = 1000
_MB = _N // _BM


def _gemm_in_body(x_ref, w_ref, o_ref):
    o_ref[...] = jnp.dot(x_ref[...], w_ref[...],
                         preferred_element_type=jnp.float32)


def _gemm_in(x, W):
    return pl.pallas_call(
        _gemm_in_body,
        grid=(_NC, _MB),
        in_specs=[
            pl.BlockSpec((_BM, _F), lambda c, i: (i, 0)),
            pl.BlockSpec((_F, _FH), lambda c, i: (0, c)),
        ],
        out_specs=pl.BlockSpec((_BM, _FH), lambda c, i: (c * _MB + i, 0)),
        out_shape=jax.ShapeDtypeStruct((_NC * _N, _FH), jnp.float32),
    )(x, W)


def _gemm_mid_body(z0_ref, z1_ref, b_ref, w_ref, o_ref):
    h0 = jnp.maximum(z0_ref[...] + b_ref[0:1, :], 0.0)
    h1 = jnp.maximum(z1_ref[...] + b_ref[1:2, :], 0.0)
    o_ref[...] = (
        jnp.dot(h0, w_ref[0:_FH, :], preferred_element_type=jnp.float32)
        + jnp.dot(h1, w_ref[_FH:, :], preferred_element_type=jnp.float32))


def _gemm_mid(z, b, W):
    b2 = b.reshape(_NC, _FH)
    return pl.pallas_call(
        _gemm_mid_body,
        grid=(_NC, _MB),
        in_specs=[
            pl.BlockSpec((_BM, _FH), lambda c, i: (i, 0)),
            pl.BlockSpec((_BM, _FH), lambda c, i: (_MB + i, 0)),
            pl.BlockSpec((_NC, _FH), lambda c, i: (0, 0)),
            pl.BlockSpec((_H, _FH), lambda c, i: (0, c)),
        ],
        out_specs=pl.BlockSpec((_BM, _FH), lambda c, i: (c * _MB + i, 0)),
        out_shape=jax.ShapeDtypeStruct((_NC * _N, _FH), jnp.float32),
    )(z, z, b2, W)


def _gemm_out_body(z0_ref, z1_ref, b_ref, w_ref, b3_ref, o_ref):
    h0 = jnp.maximum(z0_ref[...] + b_ref[0:1, :], 0.0)
    h1 = jnp.maximum(z1_ref[...] + b_ref[1:2, :], 0.0)
    o_ref[...] = (
        jnp.dot(h0, w_ref[0:_FH, :], preferred_element_type=jnp.float32)
        + jnp.dot(h1, w_ref[_FH:, :], preferred_element_type=jnp.float32)
        + b3_ref[...])


def _gemm_out(z, b, W3, b3):
    b2 = b.reshape(_NC, _FH)
    b3r = b3.reshape(1, _C)
    return pl.pallas_call(
        _gemm_out_body,
        grid=(_MB,),
        in_specs=[
            pl.BlockSpec((_BM, _FH), lambda i: (i, 0)),
            pl.BlockSpec((_BM, _FH), lambda i: (_MB + i, 0)),
            pl.BlockSpec((_NC, _FH), lambda i: (0, 0)),
            pl.BlockSpec((_H, _C), lambda i: (0, 0)),
            pl.BlockSpec((1, _C), lambda i: (0, 0)),
        ],
        out_specs=pl.BlockSpec((_BM, _C), lambda i: (i, 0)),
        out_shape=jax.ShapeDtypeStruct((_N, _C), jnp.float32),
    )(z, z, b2, W3, b3r)


_mesh = plsc.VectorSubcoreMesh(core_axis_name="c", subcore_axis_name="s")


@functools.partial(
    pl.kernel,
    mesh=_mesh,
    out_type=jax.ShapeDtypeStruct((_NC * _N, _FH), jnp.float32),
    scratch_types=[
        pltpu.VMEM((_GP, _K), jnp.int32),
        pltpu.VMEM((_GP, _K), jnp.int32),
        pltpu.VMEM((_K, _FH), jnp.float32),
        pltpu.VMEM((_K, _FH), jnp.float32),
        pltpu.VMEM((_K, _FH), jnp.float32),
        pltpu.VMEM((_K, _FH), jnp.float32),
        pltpu.VMEM((_ZR, _FH), jnp.float32),
        pltpu.VMEM_SHARED((_NP, _FH), jnp.float32),
        pltpu.SemaphoreType.DMA,
        pltpu.SemaphoreType.DMA,
        pltpu.SemaphoreType.DMA,
        pltpu.SemaphoreType.DMA,
        pltpu.SemaphoreType.DMA,
    ],
)
def _spmm(srcA_hbm, srcB_hbm, dst_hbm, y_hbm, out_hbm,
          src_v, dst_v, rows0_v, rows1_v, rows2_v, rows3_v, zero_v, accum,
          sem0, sem1, sem2, sem3, zsem):
    c = lax.axis_index("c")
    s = lax.axis_index("s")

    def _stage(p):
        @pl.when(c == 0)
        def _():
            pltpu.sync_copy(srcA_hbm.at[pl.ds(s * _NG + p * _GP, _GP)],
                            src_v)

        @pl.when(c == 1)
        def _():
            pltpu.sync_copy(srcB_hbm.at[pl.ds(s * _NG + p * _GP, _GP)],
                            src_v)

        pltpu.sync_copy(dst_hbm.at[pl.ds(s * _NG + p * _GP, _GP)], dst_v)

    bufs = (rows0_v, rows1_v, rows2_v, rows3_v)
    sems = (sem0, sem1, sem2, sem3)
    _NB = len(bufs)

    def _prime():
        for b in range(_NB):
            pltpu.async_copy(y_hbm.at[src_v.at[b]], bufs[b], sems[b])

    def _edge_phase():
        def _body(j, carry):
            g = _NB * j
            for b in range(_NB):
                pltpu.make_async_copy(y_hbm.at[src_v.at[g + b]], bufs[b],
                                      sems[b]).wait()
                pltpu.sync_copy(bufs[b], accum.at[dst_v.at[g + b]],
                                add=True)
                pltpu.async_copy(y_hbm.at[src_v.at[g + b + _NB]], bufs[b],
                                 sems[b])
            return carry
        n_full = (_GP - 2 * _NB) // _NB + 1
        lax.fori_loop(0, n_full, _body, 0)

        for g in range(_NB * n_full, _GP):
            b = g % _NB
            pltpu.make_async_copy(y_hbm.at[src_v.at[g]], bufs[b],
                                  sems[b]).wait()
            pltpu.sync_copy(bufs[b], accum.at[dst_v.at[g]], add=True)
            if g + _NB < _GP:
                pltpu.async_copy(y_hbm.at[src_v.at[g + _NB]], bufs[b],
                                 sems[b])

    _stage(0)

    _prime()

    def _zb(i, carry):
        zero_v[i // 8, pl.ds((i % 8) * 16, 16)] = jnp.zeros(
            (16,), jnp.float32)
        return carry
    lax.fori_loop(0, _ZR * 8, _zb, 0)

    def _zc(i, carry):
        pltpu.async_copy(zero_v, accum.at[pl.ds(s * _RWA + i * _ZR, _ZR)],
                         zsem)
        return carry
    lax.fori_loop(0, _RWA // _ZR, _zc, 0)

    def _zw(i, carry):
        pltpu.make_async_copy(
            zero_v, accum.at[pl.ds(s * _RWA, _ZR)], zsem).wait()
        return carry
    lax.fori_loop(0, _RWA // _ZR, _zw, 0)

    plsc.subcore_barrier()

    _edge_phase()
    for p in range(1, _NPH):
        _stage(p)
        _prime()
        _edge_phase()

    plsc.subcore_barrier()

    @pl.when(s < _NS - 1)
    def _():
        pltpu.sync_copy(accum.at[pl.ds(s * _RWA, _RWA)],
                        out_hbm.at[pl.ds(c * _N + s * _RWA, _RWA)])

    @pl.when(s == _NS - 1)
    def _():
        pltpu.sync_copy(accum.at[pl.ds(s * _RWA, _RWL)],
                        out_hbm.at[pl.ds(c * _N + s * _RWA, _RWL)])


def kernel(x, edge_index, W1, b1, W2, b2, W3, b3):
    pad = _EP - _E
    pad_iota = jnp.arange(pad, dtype=jnp.int32)
    pad2 = jnp.stack([pad_iota % _N, _N + (pad_iota % 16)])
    ei = jnp.concatenate([edge_index, pad2], axis=1)
    src = ei[0].reshape(_NS * _NG, _K)
    srcB = src + _N
    dst = ei[1].reshape(_NS * _NG, _K)

    y1 = _gemm_in(x, W1)
    z1 = _spmm(src, srcB, dst, y1)
    y2 = _gemm_mid(z1, b1, W2)
    z2 = _spmm(src, srcB, dst, y2)

    return _gemm_out(z2, b2, W3, b3)
